# baseline (speedup 1.0000x reference)
import jax
import jax.numpy as jnp
from jax.experimental import pallas as pl
from jax.experimental.pallas import tpu as pltpu

_BM = 400


def _stream_kernel(a_ref, b_ref, o_ref):
    o_ref[...] = jnp.dot(a_ref[...].astype(jnp.bfloat16), b_ref[...],
                         preferred_element_type=jnp.float32)


def _stream(a, b):
    m, k = a.shape
    c = b.shape[1]
    return pl.pallas_call(
        _stream_kernel,
        grid=(m // _BM,),
        in_specs=[pl.BlockSpec((_BM, k), lambda i: (i, 0)),
                  pl.BlockSpec((k, c), lambda i: (0, 0))],
        out_specs=pl.BlockSpec((_BM, c), lambda i: (i, 0)),
        out_shape=jax.ShapeDtypeStruct((m, c), jnp.float32),
        compiler_params=pltpu.CompilerParams(
            dimension_semantics=("arbitrary",)),
    )(a, b)


def kernel(feat, adj, ppmi, *rest):
    b = jnp.ones((adj.shape[1], 128), jnp.bfloat16)
    devs = jax.devices()
    mesh = jax.sharding.Mesh(devs[:2], ("x",))
    P = jax.sharding.PartitionSpec

    def body(a_l, p_l, b_l):
        return _stream(a_l, b_l) + _stream(p_l, b_l)

    fn = jax.shard_map(body, mesh=mesh,
                       in_specs=(P("x", None), P("x", None), P()),
                       out_specs=P("x", None), check_vma=False)
    o = fn(adj, ppmi, b)
    return (o,) * 13


# speedup vs baseline: 1.4515x; 1.4515x over previous
import functools

import jax
import jax.numpy as jnp
from jax.experimental import pallas as pl
from jax.experimental.pallas import tpu as pltpu

_BM = 400
_BJ = 1000


def _layer1_kernel(hid, a_ref, feat_ref, w1_ref, wd_ref, b1_ref, u_ref,
                   bscr_ref):
    k = a_ref.shape[1]

    @pl.when(pl.program_id(0) == 0)
    def _():
        for j in range(k // _BJ):
            blk = jnp.dot(feat_ref[j * _BJ:(j + 1) * _BJ, :], w1_ref[...],
                          preferred_element_type=jnp.float32)
            bscr_ref[j * _BJ:(j + 1) * _BJ, :] = blk.astype(jnp.bfloat16)

    s = jnp.dot(a_ref[...].astype(jnp.bfloat16), bscr_ref[...],
                preferred_element_type=jnp.float32) + b1_ref[...]
    col = jax.lax.broadcasted_iota(jnp.int32, s.shape, 1)
    s = jnp.where(col >= hid, jnp.maximum(s, 0.0), s)
    u = jnp.dot(s.astype(jnp.bfloat16), wd_ref[...],
                preferred_element_type=jnp.float32)
    u_ref[...] = u.astype(jnp.bfloat16)


def _layer1(a, featb, w1, wd, b1, hid):
    m, k = a.shape
    c = w1.shape[1]
    return pl.pallas_call(
        functools.partial(_layer1_kernel, hid),
        grid=(m // _BM,),
        in_specs=[
            pl.BlockSpec((_BM, k), lambda i: (i, 0)),
            pl.BlockSpec((k, featb.shape[1]), lambda i: (0, 0)),
            pl.BlockSpec((featb.shape[1], c), lambda i: (0, 0)),
            pl.BlockSpec((c, c), lambda i: (0, 0)),
            pl.BlockSpec((1, c), lambda i: (0, 0)),
        ],
        out_specs=pl.BlockSpec((_BM, c), lambda i: (i, 0)),
        out_shape=jax.ShapeDtypeStruct((m, c), jnp.bfloat16),
        scratch_shapes=[pltpu.VMEM((k, c), jnp.bfloat16)],
        compiler_params=pltpu.CompilerParams(
            dimension_semantics=("arbitrary",)),
    )(a, featb, w1, wd, b1)


def _layer2_kernel(out, a_ref, u_ref, b2_ref, o1a_ref, o1b_ref, o2_ref,
                   o3a_ref, o3b_ref, o4_ref):
    o = jnp.dot(a_ref[...].astype(jnp.bfloat16), u_ref[...],
                preferred_element_type=jnp.float32) + b2_ref[...]
    mu_p = o[:, :out]
    o1a_ref[...] = mu_p
    o1b_ref[...] = mu_p
    o2_ref[...] = o[:, out:2 * out]
    mu_s = o[:, 2 * out:3 * out]
    o3a_ref[...] = mu_s
    o3b_ref[...] = mu_s
    o4_ref[...] = o[:, 3 * out:]


def _layer2(a, u, b2, out):
    m, k = a.shape
    c = u.shape[1]
    shp = jax.ShapeDtypeStruct((m, out), jnp.float32)
    ospec = pl.BlockSpec((_BM, out), lambda i: (i, 0))
    return pl.pallas_call(
        functools.partial(_layer2_kernel, out),
        grid=(m // _BM,),
        in_specs=[
            pl.BlockSpec((_BM, k), lambda i: (i, 0)),
            pl.BlockSpec((k, c), lambda i: (0, 0)),
            pl.BlockSpec((1, c), lambda i: (0, 0)),
        ],
        out_specs=[ospec] * 6,
        out_shape=[shp] * 6,
        compiler_params=pltpu.CompilerParams(
            dimension_semantics=("arbitrary",)),
    )(a, u, b2)


def _layer2_att_kernel(out, a_ref, u_ref, b2_ref, musl_ref, attw_ref,
                       attb_ref, o1a_ref, o1b_ref, o2_ref, o3a_ref, o3b_ref,
                       o4_ref, sh_ref):
    o = jnp.dot(a_ref[...].astype(jnp.bfloat16), u_ref[...],
                preferred_element_type=jnp.float32) + b2_ref[...]
    mu_p = o[:, :out]
    o1a_ref[...] = mu_p
    o1b_ref[...] = mu_p
    o2_ref[...] = o[:, out:2 * out]
    mu_s_g = o[:, 2 * out:3 * out]
    o3a_ref[...] = mu_s_g
    o3b_ref[...] = mu_s_g
    o4_ref[...] = o[:, 3 * out:]

    mu_s_l = musl_ref[...]
    w = attw_ref[...]
    b = attb_ref[0, 0]
    l1 = jnp.dot(mu_s_l, w, preferred_element_type=jnp.float32) + b
    l2 = jnp.dot(mu_s_g, w, preferred_element_type=jnp.float32) + b
    z = jnp.maximum(l1, l2)
    e1 = jnp.exp(l1 - z)
    e2 = jnp.exp(l2 - z)
    sh_ref[...] = (e1 * mu_s_l + e2 * mu_s_g) / (e1 + e2)


def _layer2_att(a, u, b2, mu_s_l, att_w, att_b, out):
    m, k = a.shape
    c = u.shape[1]
    shp = jax.ShapeDtypeStruct((m, out), jnp.float32)
    ospec = pl.BlockSpec((_BM, out), lambda i: (i, 0))
    return pl.pallas_call(
        functools.partial(_layer2_att_kernel, out),
        grid=(m // _BM,),
        in_specs=[
            pl.BlockSpec((_BM, k), lambda i: (i, 0)),
            pl.BlockSpec((k, c), lambda i: (0, 0)),
            pl.BlockSpec((1, c), lambda i: (0, 0)),
            pl.BlockSpec((_BM, out), lambda i: (i, 0)),
            pl.BlockSpec((out, 1), lambda i: (0, 0)),
            pl.BlockSpec((1, 1), lambda i: (0, 0)),
        ],
        out_specs=[ospec] * 7,
        out_shape=[shp] * 7,
        compiler_params=pltpu.CompilerParams(
            dimension_semantics=("arbitrary",)),
    )(a, u, b2, mu_s_l, att_w, att_b)


def _block_diag(w_top, w_bot):
    top = jnp.concatenate([w_top, jnp.zeros_like(w_bot)], axis=1)
    bot = jnp.concatenate([jnp.zeros_like(w_top), w_bot], axis=1)
    return jnp.concatenate([top, bot], axis=0)


def kernel(feat, adj, ppmi,
           pl_W1, pl_b1, pl_W2, pl_b2, pl_W3, pl_b3,
           pg_W1, pg_b1, pg_W2, pg_b2, pg_W3, pg_b3,
           sl_W1, sl_b1, sl_W2, sl_b2, sl_W3, sl_b3,
           sg_W1, sg_b1, sg_W2, sg_b2, sg_W3, sg_b3,
           att_W, att_b):
    bf = jnp.bfloat16
    hid = pl_W1.shape[1]
    out = pl_W2.shape[1]
    featb = feat.astype(bf)

    w1_adj = jnp.concatenate([pl_W1, sl_W1], axis=1).astype(bf)
    w1_ppmi = jnp.concatenate([pg_W1, sg_W1], axis=1).astype(bf)
    b1_adj = jnp.concatenate([pl_b1, sl_b1]).reshape(1, -1)
    b1_ppmi = jnp.concatenate([pg_b1, sg_b1]).reshape(1, -1)
    wd_adj = _block_diag(jnp.concatenate([pl_W2, pl_W3], axis=1),
                         jnp.concatenate([sl_W2, sl_W3], axis=1)).astype(bf)
    wd_ppmi = _block_diag(jnp.concatenate([pg_W2, pg_W3], axis=1),
                          jnp.concatenate([sg_W2, sg_W3], axis=1)).astype(bf)
    b2_adj = jnp.concatenate([pl_b2, pl_b3, sl_b2, sl_b3]).reshape(1, -1)
    b2_ppmi = jnp.concatenate([pg_b2, pg_b3, sg_b2, sg_b3]).reshape(1, -1)

    u_adj = _layer1(adj, featb, w1_adj, wd_adj, b1_adj, hid)
    u_ppmi = _layer1(ppmi, featb, w1_ppmi, wd_ppmi, b1_ppmi, hid)

    (mu_p_l, mu_p_l2, logvar_p_l,
     mu_s_l, mu_s_l2, logvar_s_l) = _layer2(adj, u_adj, b2_adj, out)
    (mu_p_g, mu_p_g2, logvar_p_g,
     mu_s_g, mu_s_g2, logvar_s_g, shared_emb) = _layer2_att(
        ppmi, u_ppmi, b2_ppmi, mu_s_l, att_W, att_b.reshape(1, 1), out)

    return (mu_p_l, mu_p_l2, logvar_p_l,
            mu_p_g, mu_p_g2, logvar_p_g,
            mu_s_l, mu_s_l2, logvar_s_l,
            mu_s_g, mu_s_g2, logvar_s_g,
            shared_emb)


# speedup vs baseline: 1.4657x; 1.0098x over previous
import functools

import jax
import jax.numpy as jnp
from jax.experimental import pallas as pl
from jax.experimental.pallas import tpu as pltpu

_BM = 400
_BJ = 1000


def _layer1_kernel(hid, a_ref, feat_ref, w1_ref, wdt_ref, wdb_ref, b1_ref,
                   u_ref, bscr_ref):
    k = a_ref.shape[1]

    @pl.when(pl.program_id(0) == 0)
    def _():
        for j in range(k // _BJ):
            blk = jnp.dot(feat_ref[j * _BJ:(j + 1) * _BJ, :], w1_ref[...],
                          preferred_element_type=jnp.float32)
            bscr_ref[j * _BJ:(j + 1) * _BJ, :] = blk.astype(jnp.bfloat16)

    s = jnp.dot(a_ref[...].astype(jnp.bfloat16), bscr_ref[...],
                preferred_element_type=jnp.float32) + b1_ref[...]
    s1 = s[:, :hid].astype(jnp.bfloat16)
    s2 = jnp.maximum(s[:, hid:], 0.0).astype(jnp.bfloat16)
    c2 = wdt_ref.shape[1]
    u_ref[:, :c2] = jnp.dot(s1, wdt_ref[...],
                            preferred_element_type=jnp.float32
                            ).astype(jnp.bfloat16)
    u_ref[:, c2:] = jnp.dot(s2, wdb_ref[...],
                            preferred_element_type=jnp.float32
                            ).astype(jnp.bfloat16)


def _layer1(a, featb, w1, wd_top, wd_bot, b1, hid):
    m, k = a.shape
    c = w1.shape[1]
    cu = wd_top.shape[1] + wd_bot.shape[1]
    return pl.pallas_call(
        functools.partial(_layer1_kernel, hid),
        grid=(m // _BM,),
        in_specs=[
            pl.BlockSpec((_BM, k), lambda i: (i, 0)),
            pl.BlockSpec((k, featb.shape[1]), lambda i: (0, 0)),
            pl.BlockSpec((featb.shape[1], c), lambda i: (0, 0)),
            pl.BlockSpec(wd_top.shape, lambda i: (0, 0)),
            pl.BlockSpec(wd_bot.shape, lambda i: (0, 0)),
            pl.BlockSpec((1, c), lambda i: (0, 0)),
        ],
        out_specs=pl.BlockSpec((_BM, cu), lambda i: (i, 0)),
        out_shape=jax.ShapeDtypeStruct((m, cu), jnp.bfloat16),
        scratch_shapes=[pltpu.VMEM((k, c), jnp.bfloat16)],
        compiler_params=pltpu.CompilerParams(
            dimension_semantics=("arbitrary",)),
    )(a, featb, w1, wd_top, wd_bot, b1)


def _layer2_kernel(out, a_ref, u_ref, b2_ref, o1_ref, o2_ref, o3_ref, o4_ref):
    o = jnp.dot(a_ref[...].astype(jnp.bfloat16), u_ref[...],
                preferred_element_type=jnp.float32) + b2_ref[...]
    o1_ref[...] = o[:, :out]
    o2_ref[...] = o[:, out:2 * out]
    o3_ref[...] = o[:, 2 * out:3 * out]
    o4_ref[...] = o[:, 3 * out:]


def _layer2(a, u, b2, out):
    m, k = a.shape
    c = u.shape[1]
    shp = jax.ShapeDtypeStruct((m, out), jnp.float32)
    ospec = pl.BlockSpec((_BM, out), lambda i: (i, 0))
    return pl.pallas_call(
        functools.partial(_layer2_kernel, out),
        grid=(m // _BM,),
        in_specs=[
            pl.BlockSpec((_BM, k), lambda i: (i, 0)),
            pl.BlockSpec((k, c), lambda i: (0, 0)),
            pl.BlockSpec((1, c), lambda i: (0, 0)),
        ],
        out_specs=[ospec] * 4,
        out_shape=[shp] * 4,
        compiler_params=pltpu.CompilerParams(
            dimension_semantics=("arbitrary",)),
    )(a, u, b2)


def _layer2_att_kernel(out, a_ref, u_ref, b2_ref, musl_ref, attw_ref,
                       attb_ref, o1_ref, o2_ref, o3_ref, o4_ref, sh_ref):
    o = jnp.dot(a_ref[...].astype(jnp.bfloat16), u_ref[...],
                preferred_element_type=jnp.float32) + b2_ref[...]
    o1_ref[...] = o[:, :out]
    o2_ref[...] = o[:, out:2 * out]
    mu_s_g = o[:, 2 * out:3 * out]
    o3_ref[...] = mu_s_g
    o4_ref[...] = o[:, 3 * out:]

    mu_s_l = musl_ref[...]
    w = attw_ref[...]
    b = attb_ref[0, 0]
    l1 = jnp.dot(mu_s_l, w, preferred_element_type=jnp.float32) + b
    l2 = jnp.dot(mu_s_g, w, preferred_element_type=jnp.float32) + b
    z = jnp.maximum(l1, l2)
    e1 = jnp.exp(l1 - z)
    e2 = jnp.exp(l2 - z)
    sh_ref[...] = (e1 * mu_s_l + e2 * mu_s_g) / (e1 + e2)


def _layer2_att(a, u, b2, mu_s_l, att_w, att_b, out):
    m, k = a.shape
    c = u.shape[1]
    shp = jax.ShapeDtypeStruct((m, out), jnp.float32)
    ospec = pl.BlockSpec((_BM, out), lambda i: (i, 0))
    return pl.pallas_call(
        functools.partial(_layer2_att_kernel, out),
        grid=(m // _BM,),
        in_specs=[
            pl.BlockSpec((_BM, k), lambda i: (i, 0)),
            pl.BlockSpec((k, c), lambda i: (0, 0)),
            pl.BlockSpec((1, c), lambda i: (0, 0)),
            pl.BlockSpec((_BM, out), lambda i: (i, 0)),
            pl.BlockSpec((out, 1), lambda i: (0, 0)),
            pl.BlockSpec((1, 1), lambda i: (0, 0)),
        ],
        out_specs=[ospec] * 5,
        out_shape=[shp] * 5,
        compiler_params=pltpu.CompilerParams(
            dimension_semantics=("arbitrary",)),
    )(a, u, b2, mu_s_l, att_w, att_b)


def kernel(feat, adj, ppmi,
           pl_W1, pl_b1, pl_W2, pl_b2, pl_W3, pl_b3,
           pg_W1, pg_b1, pg_W2, pg_b2, pg_W3, pg_b3,
           sl_W1, sl_b1, sl_W2, sl_b2, sl_W3, sl_b3,
           sg_W1, sg_b1, sg_W2, sg_b2, sg_W3, sg_b3,
           att_W, att_b):
    bf = jnp.bfloat16
    hid = pl_W1.shape[1]
    out = pl_W2.shape[1]
    featb = feat.astype(bf)

    w1_adj = jnp.concatenate([pl_W1, sl_W1], axis=1).astype(bf)
    w1_ppmi = jnp.concatenate([pg_W1, sg_W1], axis=1).astype(bf)
    b1_adj = jnp.concatenate([pl_b1, sl_b1]).reshape(1, -1)
    b1_ppmi = jnp.concatenate([pg_b1, sg_b1]).reshape(1, -1)
    wdt_adj = jnp.concatenate([pl_W2, pl_W3], axis=1).astype(bf)
    wdb_adj = jnp.concatenate([sl_W2, sl_W3], axis=1).astype(bf)
    wdt_ppmi = jnp.concatenate([pg_W2, pg_W3], axis=1).astype(bf)
    wdb_ppmi = jnp.concatenate([sg_W2, sg_W3], axis=1).astype(bf)
    b2_adj = jnp.concatenate([pl_b2, pl_b3, sl_b2, sl_b3]).reshape(1, -1)
    b2_ppmi = jnp.concatenate([pg_b2, pg_b3, sg_b2, sg_b3]).reshape(1, -1)

    u_adj = _layer1(adj, featb, w1_adj, wdt_adj, wdb_adj, b1_adj, hid)
    u_ppmi = _layer1(ppmi, featb, w1_ppmi, wdt_ppmi, wdb_ppmi, b1_ppmi, hid)

    mu_p_l, logvar_p_l, mu_s_l, logvar_s_l = _layer2(adj, u_adj, b2_adj, out)
    mu_p_g, logvar_p_g, mu_s_g, logvar_s_g, shared_emb = _layer2_att(
        ppmi, u_ppmi, b2_ppmi, mu_s_l, att_W, att_b.reshape(1, 1), out)

    return (mu_p_l, mu_p_l, logvar_p_l,
            mu_p_g, mu_p_g, logvar_p_g,
            mu_s_l, mu_s_l, logvar_s_l,
            mu_s_g, mu_s_g, logvar_s_g,
            shared_emb)
